# baseline (speedup 1.0000x reference)
import functools

import jax
import jax.numpy as jnp
from jax import lax
from jax.experimental import pallas as pl
from jax.experimental.pallas import tpu as pltpu
from jax.experimental.pallas import tpu_sc as plsc

_NC = 2
_NS = 16
_NW = _NC * _NS
_C = 8
_R = 3


@functools.lru_cache(maxsize=None)
def _build(n, v, d):
    assert n % (_NW * _C) == 0
    bpw = n // _NW
    nchunk = bpw // _C
    nfull = (nchunk - _R) // _R
    tail = nchunk - _R * nfull

    mesh = plsc.VectorSubcoreMesh(core_axis_name="c", subcore_axis_name="s")

    @functools.partial(
        pl.kernel,
        out_type=jax.ShapeDtypeStruct((n, d), jnp.float32),
        mesh=mesh,
        scratch_types=[
            pltpu.VMEM((bpw,), jnp.int32),
            [pltpu.VMEM((_C, d), jnp.float32) for _ in range(_R)],
            [pltpu.SemaphoreType.DMA for _ in range(_R)],
            [pltpu.SemaphoreType.DMA for _ in range(_R)],
        ],
    )
    def emb(idx_hbm, table_hbm, out_hbm, idx_v, rows, gs, os):
        wid = lax.axis_index("s") * _NC + lax.axis_index("c")
        base = wid * bpw
        pltpu.sync_copy(idx_hbm.at[pl.ds(base, bpw)], idx_v)

        def gather(chunk, b):
            off = chunk * _C
            pltpu.async_copy(table_hbm.at[idx_v.at[pl.ds(off, _C)]],
                             rows[b], gs[b])

        def wait_gather(b):
            pltpu.make_async_copy(table_hbm.at[pl.ds(0, _C)], rows[b],
                                  gs[b]).wait()

        def put(chunk, b):
            off = chunk * _C
            pltpu.async_copy(rows[b], out_hbm.at[pl.ds(base + off, _C)],
                             os[b])

        def wait_put(b):
            pltpu.make_async_copy(rows[b], out_hbm.at[pl.ds(base, _C)],
                                  os[b]).wait()

        for b in range(_R):
            gather(b, b)

        def body(g, carry):
            for b in range(_R):
                i = g * _R + b
                wait_gather(b)
                put(i, b)
            for b in range(_R):
                wait_put(b)
                gather(g * _R + b + _R, b)
            return carry

        lax.fori_loop(0, nfull, body, 0)

        for t in range(tail):
            i = nfull * _R + t
            b = i % _R
            wait_gather(b)
            put(i, b)
            if i + _R < nchunk:
                wait_put(b)
                gather(i + _R, b)
        for b in range(_R):
            wait_put(b)

    return emb


def kernel(indices, embedding_weight):
    b, t = indices.shape
    v, d = embedding_weight.shape
    flat = indices.reshape(-1).astype(jnp.int32)
    out = _build(flat.shape[0], v, d)(flat, embedding_weight)
    return out.reshape(b, t, d)
